# baseline (speedup 1.0000x reference)
import functools

import jax
import jax.numpy as jnp
from jax import lax
from jax.experimental import pallas as pl
from jax.experimental.pallas import tpu as pltpu
from jax.experimental.pallas import tpu_sc as plsc

P, L, Q = 10000, 1000, 3000
PL = 8
K1, K2, QPL = 80, 32, 3
ITERS = 8
ZSC = {'traffic': [1385.4059, 859.8119], 'packets': [1.4015, 0.8933], 'eq_lambda': [1350.9712, 858.3162], 'avg_pkts_lambda': [0.9117, 0.9724], 'exp_max_factor': [6.6636, 4.7151], 'pkts_lambda_on': [0.9116, 1.6513], 'avg_t_off': [1.6649, 2.3564], 'avg_t_on': [1.6649, 2.3564], 'ar_a': [0.0, 1.0], 'sigma': [0.0, 1.0], 'capacity': [27611.0918, 20090.6211], 'queue_size': [30259.1055, 21410.0957]}

NC, NS = 2, 16
NW = NC * NS
CH = 128


def _sc_mesh():
    return plsc.VectorSubcoreMesh(core_axis_name="c", subcore_axis_name="s",
                                  num_cores=NC, num_subcores=NS)


def _wid():
    return lax.axis_index("s") * NC + lax.axis_index("c")


def _sc_gather_pair(vq, vl, n_chunks, interpret=False):
    b = n_chunks * CH

    @functools.partial(
        pl.kernel,
        out_type=(jax.ShapeDtypeStruct((NW * b, 32), jnp.float32),
                  jax.ShapeDtypeStruct((NW * b, 32), jnp.float32)),
        mesh=_sc_mesh(),
        scratch_types=[pltpu.VMEM((n_chunks, CH), jnp.int32),
                       pltpu.VMEM((b, 32), jnp.float32),
                       pltpu.SemaphoreType.DMA],
        compiler_params=pltpu.CompilerParams(use_tc_tiling_on_sc=False),
        interpret=interpret,
    )
    def k(qtab, ltab, qidx, lidx, qout, lout, idx_v, rows_v, sem):
        wid = _wid()
        base = wid * b
        for tab, idx, out in ((qtab, qidx, qout), (ltab, lidx, lout)):
            pltpu.sync_copy(idx.at[wid], idx_v)
            descs = []
            for c in range(n_chunks):
                descs.append(pltpu.async_copy(
                    tab.at[idx_v.at[c]], rows_v.at[pl.ds(c * CH, CH)], sem))
            for d in descs:
                d.wait()
            pltpu.sync_copy(rows_v, out.at[pl.ds(base, b)])

    return k


def _sc_gather_one(n_chunks, ch, interpret=False):
    b = n_chunks * ch

    @functools.partial(
        pl.kernel,
        out_type=jax.ShapeDtypeStruct((NW * b, 32), jnp.float32),
        mesh=_sc_mesh(),
        scratch_types=[pltpu.VMEM((n_chunks, ch), jnp.int32),
                       pltpu.VMEM((b, 32), jnp.float32),
                       pltpu.SemaphoreType.DMA],
        compiler_params=pltpu.CompilerParams(use_tc_tiling_on_sc=False),
        interpret=interpret,
    )
    def k(tab, idx, out, idx_v, rows_v, sem):
        wid = _wid()
        base = wid * b
        pltpu.sync_copy(idx.at[wid], idx_v)
        descs = []
        for c in range(n_chunks):
            descs.append(pltpu.async_copy(
                tab.at[idx_v.at[c]], rows_v.at[pl.ds(c * ch, ch)], sem))
        for d in descs:
            d.wait()
        pltpu.sync_copy(rows_v, out.at[pl.ds(base, b)])

    return k


def _sc_gather_sum(n_chunks, interpret=False):
    b = n_chunks * CH
    GQ = b // K2

    @functools.partial(
        pl.kernel,
        out_type=jax.ShapeDtypeStruct((NW * GQ, 32), jnp.float32),
        mesh=_sc_mesh(),
        scratch_types=[pltpu.VMEM((n_chunks, CH), jnp.int32),
                       pltpu.VMEM((b, 32), jnp.float32),
                       pltpu.VMEM((GQ, 32), jnp.float32),
                       pltpu.SemaphoreType.DMA],
        compiler_params=pltpu.CompilerParams(use_tc_tiling_on_sc=False,
                                             needs_layout_passes=False),
        interpret=interpret,
    )
    def k(tab, idx, out, idx_v, rows_v, sum_v, sem):
        wid = _wid()
        pltpu.sync_copy(idx.at[wid], idx_v)
        descs = []
        for c in range(n_chunks):
            descs.append(pltpu.async_copy(
                tab.at[idx_v.at[c]], rows_v.at[pl.ds(c * CH, CH)], sem))
        for d in descs:
            d.wait()

        def q_body(q, carry):
            base = q * K2
            a0 = jnp.zeros((16,), jnp.float32)
            a1 = jnp.zeros((16,), jnp.float32)
            for kk in range(K2):
                a0 = a0 + rows_v[base + kk, pl.ds(0, 16)]
                a1 = a1 + rows_v[base + kk, pl.ds(16, 16)]
            sum_v[q, pl.ds(0, 16)] = a0
            sum_v[q, pl.ds(16, 16)] = a1
            return carry

        lax.fori_loop(0, GQ, q_body, 0)
        pltpu.sync_copy(sum_v, out.at[pl.ds(wid * GQ, GQ)])

    return k


LB_W = 32
CB_W = 2512
CAP_B = NW * CB_W


def _sc_prep(interpret=False):
    @functools.partial(
        pl.kernel,
        out_type=(jax.ShapeDtypeStruct((NW * LB_W, 16), jnp.float32),
                  jax.ShapeDtypeStruct((CAP_B,), jnp.float32)),
        mesh=_sc_mesh(),
        scratch_types=[pltpu.VMEM((P,), jnp.float32),
                       pltpu.VMEM((L,), jnp.float32),
                       pltpu.VMEM((LB_W * K1,), jnp.int32),
                       pltpu.VMEM((CB_W,), jnp.int32),
                       pltpu.VMEM((LB_W, 16), jnp.float32),
                       pltpu.VMEM((CB_W,), jnp.float32)],
        compiler_params=pltpu.CompilerParams(needs_layout_passes=False),
        interpret=interpret,
    )
    def k(traffic, cap, ptl_idx, cap_idx, loadsum16, capg,
          tr_v, cap_v, pidx_v, cidx_v, ls_v, capo_v):
        wid = _wid()
        pltpu.sync_copy(traffic, tr_v)
        pltpu.sync_copy(cap, cap_v)
        pltpu.sync_copy(ptl_idx.at[wid], pidx_v)
        pltpu.sync_copy(cap_idx.at[wid], cidx_v)

        def link_body(i, carry):
            acc = jnp.zeros((16,), jnp.float32)
            for c in range(K1 // 16):
                iv = pidx_v[pl.ds(i * K1 + c * 16, 16)]
                acc = acc + plsc.load_gather(tr_v, [iv])
            ls_v[i] = acc
            return carry

        lax.fori_loop(0, LB_W, link_body, 0)
        pltpu.sync_copy(ls_v, loadsum16.at[pl.ds(wid * LB_W, LB_W)])

        def cap_body(c, carry):
            iv = cidx_v[pl.ds(c * 16, 16)]
            capo_v[pl.ds(c * 16, 16)] = plsc.load_gather(cap_v, [iv])
            return carry

        lax.fori_loop(0, CB_W // 16, cap_body, 0)
        pltpu.sync_copy(capo_v, capg.at[pl.ds(wid * CB_W, CB_W)])

    return k


def _relu(x):
    return jnp.maximum(x, 0.0)


def _tc_path_encoder(interpret=False):
    bp = 2000

    def body(f_ref, m_ref, mu_ref, sd_ref, w1_ref, b1_ref,
             w2_ref, b2_ref, out_ref):
        f = (f_ref[...] - mu_ref[...]) / sd_ref[...]
        oh = (m_ref[...] == lax.broadcasted_iota(jnp.int32, (bp, 7), 1))
        oh = oh.astype(jnp.float32)
        x = jnp.concatenate([f[:, 0:2], oh, f[:, 2:10]], axis=1)
        h1 = _relu(x @ w1_ref[...] + b1_ref[...])
        out_ref[...] = _relu(h1 @ w2_ref[...] + b2_ref[...])

    return pl.pallas_call(
        body,
        grid=(P // bp,),
        in_specs=[
            pl.BlockSpec((bp, 10), lambda i: (i, 0)),
            pl.BlockSpec((bp, 1), lambda i: (i, 0)),
            pl.BlockSpec((1, 10), lambda i: (0, 0)),
            pl.BlockSpec((1, 10), lambda i: (0, 0)),
            pl.BlockSpec((17, 32), lambda i: (0, 0)),
            pl.BlockSpec((1, 32), lambda i: (0, 0)),
            pl.BlockSpec((32, 32), lambda i: (0, 0)),
            pl.BlockSpec((1, 32), lambda i: (0, 0)),
        ],
        out_specs=pl.BlockSpec((bp, 32), lambda i: (i, 0)),
        out_shape=jax.ShapeDtypeStruct((P, 32), jnp.float32),
        interpret=interpret,
    )


def _tc_link_encoder(interpret=False):
    def body(ls16_ref, cap_ref, pol_ref, w1_ref, b1_ref, w2_ref,
             b2_ref, wxl_ref, out_ref, outw_ref):
        load = (jnp.sum(ls16_ref[...], axis=1, keepdims=True)
                / cap_ref[...])
        oh = (pol_ref[...] == lax.broadcasted_iota(jnp.int32, (L, 4), 1))
        oh = oh.astype(jnp.float32)
        x = jnp.concatenate([load, oh], axis=1)
        h1 = _relu(x @ w1_ref[...] + b1_ref[...])
        ls = _relu(h1 @ w2_ref[...] + b2_ref[...])
        out_ref[...] = ls
        outw_ref[...] = ls @ wxl_ref[...]

    return pl.pallas_call(
        body,
        in_specs=[pl.BlockSpec((L, 16), lambda: (0, 0)),
                  pl.BlockSpec((L, 1), lambda: (0, 0)),
                  pl.BlockSpec((L, 1), lambda: (0, 0)),
                  pl.BlockSpec((5, 32), lambda: (0, 0)),
                  pl.BlockSpec((1, 32), lambda: (0, 0)),
                  pl.BlockSpec((32, 32), lambda: (0, 0)),
                  pl.BlockSpec((1, 32), lambda: (0, 0)),
                  pl.BlockSpec((32, 32), lambda: (0, 0))],
        out_specs=[pl.BlockSpec((L, 32), lambda: (0, 0)),
                   pl.BlockSpec((L, 32), lambda: (0, 0))],
        out_shape=[jax.ShapeDtypeStruct((L, 32), jnp.float32),
                   jax.ShapeDtypeStruct((L, 32), jnp.float32)],
        interpret=interpret,
    )


def _tc_queue_encoder(interpret=False):
    mu, sdv = ZSC['queue_size']

    def body(qs_ref, pri_ref, w_ref, w1_ref, b1_ref,
             w2_ref, b2_ref, wxq_ref, wxl_ref, out_ref, outw_ref, outlw_ref):
        qs = (qs_ref[...] - mu) / sdv
        oh = (pri_ref[...] == lax.broadcasted_iota(jnp.int32, (Q, 3), 1))
        oh = oh.astype(jnp.float32)
        x = jnp.concatenate([qs, oh, w_ref[...]], axis=1)
        h1 = _relu(x @ w1_ref[...] + b1_ref[...])
        s = _relu(h1 @ w2_ref[...] + b2_ref[...])
        out_ref[...] = s
        outw_ref[...] = s @ wxq_ref[...]
        outlw_ref[...] = s @ wxl_ref[...]

    return pl.pallas_call(
        body,
        in_specs=[pl.BlockSpec((Q, 1), lambda: (0, 0)),
                  pl.BlockSpec((Q, 1), lambda: (0, 0)),
                  pl.BlockSpec((Q, 1), lambda: (0, 0)),
                  pl.BlockSpec((5, 32), lambda: (0, 0)),
                  pl.BlockSpec((1, 32), lambda: (0, 0)),
                  pl.BlockSpec((32, 32), lambda: (0, 0)),
                  pl.BlockSpec((1, 32), lambda: (0, 0)),
                  pl.BlockSpec((32, 32), lambda: (0, 0)),
                  pl.BlockSpec((32, 32), lambda: (0, 0))],
        out_specs=[pl.BlockSpec((Q, 32), lambda: (0, 0)),
                   pl.BlockSpec((Q, 32), lambda: (0, 0)),
                   pl.BlockSpec((Q, 32), lambda: (0, 0))],
        out_shape=[jax.ShapeDtypeStruct((Q, 32), jnp.float32),
                   jax.ShapeDtypeStruct((Q, 32), jnp.float32),
                   jax.ShapeDtypeStruct((Q, 32), jnp.float32)],
        interpret=interpret,
    )


def _tc_path_rnn(interpret=False):
    P4 = P // 4

    def body(qgw_ref, lgw_ref, ps_ref, whb_ref, b_ref, out_ref):
        h = ps_ref[...]
        out_ref[0] = h
        whb, b = whb_ref[...], b_ref[...]
        for t in range(PL):
            h = jnp.tanh(qgw_ref[t] + lgw_ref[t] + h @ whb + b)
            out_ref[t + 1] = h

    return pl.pallas_call(
        body,
        in_specs=[pl.BlockSpec((PL, P4, 128), lambda: (0, 0, 0)),
                  pl.BlockSpec((PL, P4, 128), lambda: (0, 0, 0)),
                  pl.BlockSpec((P4, 128), lambda: (0, 0)),
                  pl.BlockSpec((128, 128), lambda: (0, 0)),
                  pl.BlockSpec((1, 128), lambda: (0, 0))],
        out_specs=pl.BlockSpec((PL + 1, P4, 128), lambda: (0, 0, 0)),
        out_shape=jax.ShapeDtypeStruct((PL + 1, P4, 128), jnp.float32),
        interpret=interpret,
    )


def _tc_queue_update(interpret=False):
    bq = 600

    def body(pg_ref, qs_ref, wx_ref, wh_ref, b_ref, wxq_ref, wxl_ref,
             out_ref, outw_ref, outlw_ref):
        s = jnp.tanh(pg_ref[...] @ wx_ref[...] + qs_ref[...] @ wh_ref[...]
                     + b_ref[...])
        out_ref[...] = s
        outw_ref[...] = s @ wxq_ref[...]
        outlw_ref[...] = s @ wxl_ref[...]

    return pl.pallas_call(
        body,
        grid=(Q // bq,),
        in_specs=[pl.BlockSpec((bq, 32), lambda i: (i, 0)),
                  pl.BlockSpec((bq, 32), lambda i: (i, 0)),
                  pl.BlockSpec((32, 32), lambda i: (0, 0)),
                  pl.BlockSpec((32, 32), lambda i: (0, 0)),
                  pl.BlockSpec((1, 32), lambda i: (0, 0)),
                  pl.BlockSpec((32, 32), lambda i: (0, 0)),
                  pl.BlockSpec((32, 32), lambda i: (0, 0))],
        out_specs=[pl.BlockSpec((bq, 32), lambda i: (i, 0)),
                   pl.BlockSpec((bq, 32), lambda i: (i, 0)),
                   pl.BlockSpec((bq, 32), lambda i: (i, 0))],
        out_shape=[jax.ShapeDtypeStruct((Q, 32), jnp.float32),
                   jax.ShapeDtypeStruct((Q, 32), jnp.float32),
                   jax.ShapeDtypeStruct((Q, 32), jnp.float32)],
        interpret=interpret,
    )


def _tc_link_rnn(interpret=False):
    def body(qgw_ref, ls_ref, wh_ref, b_ref, wxl_ref, out_ref, outw_ref):
        h = ls_ref[...]
        wh, b = wh_ref[...], b_ref[...]
        for t in range(QPL):
            h = jnp.tanh(qgw_ref[:, t, :] + h @ wh + b)
        out_ref[...] = h
        outw_ref[...] = h @ wxl_ref[...]

    return pl.pallas_call(
        body,
        in_specs=[pl.BlockSpec((L, QPL, 32), lambda: (0, 0, 0)),
                  pl.BlockSpec((L, 32), lambda: (0, 0)),
                  pl.BlockSpec((32, 32), lambda: (0, 0)),
                  pl.BlockSpec((1, 32), lambda: (0, 0)),
                  pl.BlockSpec((32, 32), lambda: (0, 0))],
        out_specs=[pl.BlockSpec((L, 32), lambda: (0, 0)),
                   pl.BlockSpec((L, 32), lambda: (0, 0))],
        out_shape=[jax.ShapeDtypeStruct((L, 32), jnp.float32),
                   jax.ShapeDtypeStruct((L, 32), jnp.float32)],
        interpret=interpret,
    )


def _tc_readout(interpret=False):
    bp = 1000

    def body(h_ref, cap_ref, len_ref, tr_ref, pk_ref, w1_ref, b1_ref,
             w2_ref, b2_ref, w3_ref, b3_ref, out_ref):
        w1, b1 = w1_ref[...], b1_ref[...]
        w2, b2 = w2_ref[...], b2_ref[...]
        w3, b3 = w3_ref[...], b3_ref[...]
        ln = len_ref[...]
        qd = jnp.zeros((bp, 1), jnp.float32)
        sic = jnp.zeros((bp, 1), jnp.float32)
        for t in range(PL):
            o = _relu(h_ref[t] @ w1 + b1)
            o = _relu(o @ w2 + b2)
            o = o @ w3 + b3
            ic = 1.0 / cap_ref[:, t:t + 1]
            m = jnp.where(ln > t, 1.0, 0.0)
            qd = qd + m * o * ic
            sic = sic + m * ic
        out_ref[...] = qd + (tr_ref[...] / pk_ref[...]) * sic

    return pl.pallas_call(
        body,
        grid=(P // bp,),
        in_specs=[pl.BlockSpec((PL, bp, 32), lambda i: (0, i, 0)),
                  pl.BlockSpec((bp, PL), lambda i: (i, 0)),
                  pl.BlockSpec((bp, 1), lambda i: (i, 0)),
                  pl.BlockSpec((bp, 1), lambda i: (i, 0)),
                  pl.BlockSpec((bp, 1), lambda i: (i, 0)),
                  pl.BlockSpec((32, 16), lambda i: (0, 0)),
                  pl.BlockSpec((1, 16), lambda i: (0, 0)),
                  pl.BlockSpec((16, 16), lambda i: (0, 0)),
                  pl.BlockSpec((1, 16), lambda i: (0, 0)),
                  pl.BlockSpec((16, 1), lambda i: (0, 0)),
                  pl.BlockSpec((1, 1), lambda i: (0, 0))],
        out_specs=pl.BlockSpec((bp, 1), lambda i: (i, 0)),
        out_shape=jax.ShapeDtypeStruct((P, 1), jnp.float32),
        interpret=interpret,
    )


def _pad_to(x, n):
    return jnp.concatenate(
        [x, jnp.zeros((n - x.shape[0],) + x.shape[1:], x.dtype)], axis=0)


def kernel(traffic, packets, eq_lambda, avg_pkts_lambda, exp_max_factor, pkts_lambda_on, avg_t_off, avg_t_on, ar_a, sigma, capacity, queue_size, weight, W_pe1, b_pe1, W_pe2, b_pe2, W_le1, b_le1, W_le2, b_le2, W_qe1, b_qe1, W_qe2, b_qe2, Wx_p, b_p, Wh_p, Wx_q, b_q, Wh_q, Wx_l, b_l, Wh_l, W_r1, b_r1, W_r2, b_r2, W_r3, b_r3, length, model, policy, priority, queue_to_path, link_to_path, path_to_link, path_to_queue, queue_to_link):
    f32 = jnp.float32
    i32 = jnp.int32

    n_pp = 20
    qtp_idx = _pad_to(queue_to_path.T.reshape(-1).astype(i32),
                      NW * n_pp * CH).reshape(NW, n_pp, CH)
    ltp_idx = _pad_to(link_to_path.T.reshape(-1).astype(i32),
                      NW * n_pp * CH).reshape(NW, n_pp, CH)
    n_q = 24
    p2q_flat = (path_to_queue[:, :, 1] * P
                + path_to_queue[:, :, 0]).reshape(-1).astype(i32)
    p2q_idx = _pad_to(p2q_flat, NW * n_q * CH).reshape(NW, n_q, CH)
    qtl_idx = _pad_to(queue_to_link.reshape(-1).astype(i32),
                      NW * 96).reshape(NW, 1, 96)
    ptl_idx = _pad_to(path_to_link[:, :, 0].reshape(-1).astype(i32),
                      NW * LB_W * K1).reshape(NW, LB_W * K1)
    cap_idx = _pad_to(link_to_path.reshape(-1).astype(i32),
                      CAP_B).reshape(NW, CB_W)

    loadsum16, capg_flat = _sc_prep()(traffic.reshape(-1), capacity.reshape(-1),
                                      ptl_idx, cap_idx)
    capg = capg_flat[:P * PL].reshape(P, PL)

    feats10 = jnp.concatenate([traffic, packets, eq_lambda, avg_pkts_lambda,
                               exp_max_factor, pkts_lambda_on, avg_t_off,
                               avg_t_on, ar_a, sigma], axis=1)
    znames = ('traffic', 'packets', 'eq_lambda', 'avg_pkts_lambda',
              'exp_max_factor', 'pkts_lambda_on', 'avg_t_off', 'avg_t_on',
              'ar_a', 'sigma')
    zmu = jnp.array([[ZSC[n][0] for n in znames]], f32)
    zsd = jnp.array([[ZSC[n][1] for n in znames]], f32)
    Wxq_p, Wxl_p = Wx_p[0:32], Wx_p[32:64]
    path_state = _tc_path_encoder()(
        feats10, model.reshape(P, 1).astype(i32), zmu, zsd,
        W_pe1, b_pe1.reshape(1, 32), W_pe2, b_pe2.reshape(1, 32))
    link_state, link_w = _tc_link_encoder()(
        loadsum16[:L], capacity, policy.reshape(L, 1).astype(i32),
        W_le1, b_le1.reshape(1, 32), W_le2, b_le2.reshape(1, 32), Wxl_p)
    queue_state, queue_w, queue_lw = _tc_queue_encoder()(
        queue_size, priority.reshape(Q, 1).astype(i32), weight,
        W_qe1, b_qe1.reshape(1, 32), W_qe2, b_qe2.reshape(1, 32),
        Wxq_p, Wx_l)

    gather_pair = _sc_gather_pair(Q, L, n_pp)
    gather_p2q = _sc_gather_sum(n_q)
    gather_qtl = _sc_gather_one(1, 96)
    path_rnn = _tc_path_rnn()
    queue_update = _tc_queue_update()
    link_rnn = _tc_link_rnn()

    b_p2 = b_p.reshape(1, 32)
    b_q2 = b_q.reshape(1, 32)
    b_l2 = b_l.reshape(1, 32)

    Whp_blk = jax.scipy.linalg.block_diag(Wh_p, Wh_p, Wh_p, Wh_p)
    b_p4 = jnp.tile(b_p, 4).reshape(1, 128)
    ps4 = path_state.reshape(P // 4, 128)

    pss_tm = None
    for _ in range(ITERS):
        qgw_f, lgw_f = gather_pair(queue_w, link_w, qtp_idx, ltp_idx)
        qgw = qgw_f[:P * PL].reshape(PL, P // 4, 128)
        lgw = lgw_f[:P * PL].reshape(PL, P // 4, 128)
        pss_tm = path_rnn(qgw, lgw, ps4, Whp_blk, b_p4)
        ps4 = pss_tm[PL]
        psum = gather_p2q(pss_tm.reshape((PL + 1) * P, 32), p2q_idx)[:Q]
        queue_state, queue_w, queue_lw = queue_update(
            psum, queue_state, Wx_q, Wh_q, b_q2, Wxq_p, Wx_l)
        qglw_f = gather_qtl(queue_lw, qtl_idx)
        qglw = qglw_f[:L * QPL].reshape(L, QPL, 32)
        link_state, link_w = link_rnn(qglw, link_state, Wh_l, b_l2, Wxl_p)

    hidden = pss_tm.reshape(PL + 1, P, 32)[1:]
    out = _tc_readout()(
        hidden, capg, length.reshape(P, 1).astype(i32), traffic, packets,
        W_r1, b_r1.reshape(1, 16), W_r2, b_r2.reshape(1, 16),
        W_r3, b_r3.reshape(1, 1))
    return out


# speedup vs baseline: 1.0543x; 1.0543x over previous
import functools

import jax
import jax.numpy as jnp
from jax import lax
from jax.experimental import pallas as pl
from jax.experimental.pallas import tpu as pltpu
from jax.experimental.pallas import tpu_sc as plsc

P, L, Q = 10000, 1000, 3000
PL = 8
K1, K2, QPL = 80, 32, 3
ITERS = 8
ZSC = {'traffic': [1385.4059, 859.8119], 'packets': [1.4015, 0.8933], 'eq_lambda': [1350.9712, 858.3162], 'avg_pkts_lambda': [0.9117, 0.9724], 'exp_max_factor': [6.6636, 4.7151], 'pkts_lambda_on': [0.9116, 1.6513], 'avg_t_off': [1.6649, 2.3564], 'avg_t_on': [1.6649, 2.3564], 'ar_a': [0.0, 1.0], 'sigma': [0.0, 1.0], 'capacity': [27611.0918, 20090.6211], 'queue_size': [30259.1055, 21410.0957]}

NC, NS = 2, 16
NW = NC * NS
CH = 128


def _sc_mesh():
    return plsc.VectorSubcoreMesh(core_axis_name="c", subcore_axis_name="s",
                                  num_cores=NC, num_subcores=NS)


def _wid():
    return lax.axis_index("s") * NC + lax.axis_index("c")


def _sc_gather_pair(vq, vl, n_chunks, interpret=False):
    b = n_chunks * CH

    @functools.partial(
        pl.kernel,
        out_type=(jax.ShapeDtypeStruct((NW * b, 32), jnp.float32),
                  jax.ShapeDtypeStruct((NW * b, 32), jnp.float32)),
        mesh=_sc_mesh(),
        scratch_types=[pltpu.VMEM((n_chunks, CH), jnp.int32),
                       pltpu.VMEM((b, 32), jnp.float32),
                       pltpu.SemaphoreType.DMA],
        compiler_params=pltpu.CompilerParams(use_tc_tiling_on_sc=False),
        interpret=interpret,
    )
    def k(qtab, ltab, qidx, lidx, qout, lout, idx_v, rows_v, sem):
        wid = _wid()
        base = wid * b
        for tab, idx, out in ((qtab, qidx, qout), (ltab, lidx, lout)):
            pltpu.sync_copy(idx.at[wid], idx_v)
            descs = []
            for c in range(n_chunks):
                descs.append(pltpu.async_copy(
                    tab.at[idx_v.at[c]], rows_v.at[pl.ds(c * CH, CH)], sem))
            for d in descs:
                d.wait()
            pltpu.sync_copy(rows_v, out.at[pl.ds(base, b)])

    return k


def _sc_gather_one(n_chunks, ch, interpret=False):
    b = n_chunks * ch

    @functools.partial(
        pl.kernel,
        out_type=jax.ShapeDtypeStruct((NW * b, 32), jnp.float32),
        mesh=_sc_mesh(),
        scratch_types=[pltpu.VMEM((n_chunks, ch), jnp.int32),
                       pltpu.VMEM((b, 32), jnp.float32),
                       pltpu.SemaphoreType.DMA],
        compiler_params=pltpu.CompilerParams(use_tc_tiling_on_sc=False),
        interpret=interpret,
    )
    def k(tab, idx, out, idx_v, rows_v, sem):
        wid = _wid()
        base = wid * b
        pltpu.sync_copy(idx.at[wid], idx_v)
        descs = []
        for c in range(n_chunks):
            descs.append(pltpu.async_copy(
                tab.at[idx_v.at[c]], rows_v.at[pl.ds(c * ch, ch)], sem))
        for d in descs:
            d.wait()
        pltpu.sync_copy(rows_v, out.at[pl.ds(base, b)])

    return k


def _sc_gather_sum(n_chunks, interpret=False):
    b = n_chunks * CH
    GQ = b // K2

    @functools.partial(
        pl.kernel,
        out_type=jax.ShapeDtypeStruct((NW * GQ, 32), jnp.float32),
        mesh=_sc_mesh(),
        scratch_types=[pltpu.VMEM((n_chunks, CH), jnp.int32),
                       pltpu.VMEM((b, 32), jnp.float32),
                       pltpu.VMEM((GQ, 32), jnp.float32),
                       pltpu.SemaphoreType.DMA],
        compiler_params=pltpu.CompilerParams(use_tc_tiling_on_sc=False,
                                             needs_layout_passes=False),
        interpret=interpret,
    )
    def k(tab, idx, out, idx_v, rows_v, sum_v, sem):
        wid = _wid()
        pltpu.sync_copy(idx.at[wid], idx_v)
        descs = []
        for c in range(n_chunks):
            descs.append(pltpu.async_copy(
                tab.at[idx_v.at[c]], rows_v.at[pl.ds(c * CH, CH)], sem))
        for d in descs:
            d.wait()

        def q_body(q, carry):
            base = q * K2
            a0 = jnp.zeros((16,), jnp.float32)
            a1 = jnp.zeros((16,), jnp.float32)
            for kk in range(K2):
                a0 = a0 + rows_v[base + kk, pl.ds(0, 16)]
                a1 = a1 + rows_v[base + kk, pl.ds(16, 16)]
            sum_v[q, pl.ds(0, 16)] = a0
            sum_v[q, pl.ds(16, 16)] = a1
            return carry

        lax.fori_loop(0, GQ, q_body, 0)
        pltpu.sync_copy(sum_v, out.at[pl.ds(wid * GQ, GQ)])

    return k


LB_W = 32
CB_W = 2512
CAP_B = NW * CB_W


def _sc_prep(interpret=False):
    @functools.partial(
        pl.kernel,
        out_type=(jax.ShapeDtypeStruct((NW * LB_W, 16), jnp.float32),
                  jax.ShapeDtypeStruct((CAP_B,), jnp.float32)),
        mesh=_sc_mesh(),
        scratch_types=[pltpu.VMEM((P,), jnp.float32),
                       pltpu.VMEM((L,), jnp.float32),
                       pltpu.VMEM((LB_W * K1,), jnp.int32),
                       pltpu.VMEM((CB_W,), jnp.int32),
                       pltpu.VMEM((LB_W, 16), jnp.float32),
                       pltpu.VMEM((CB_W,), jnp.float32)],
        compiler_params=pltpu.CompilerParams(needs_layout_passes=False),
        interpret=interpret,
    )
    def k(traffic, cap, ptl_idx, cap_idx, loadsum16, capg,
          tr_v, cap_v, pidx_v, cidx_v, ls_v, capo_v):
        wid = _wid()
        pltpu.sync_copy(traffic, tr_v)
        pltpu.sync_copy(cap, cap_v)
        pltpu.sync_copy(ptl_idx.at[wid], pidx_v)
        pltpu.sync_copy(cap_idx.at[wid], cidx_v)

        def link_body(i, carry):
            acc = jnp.zeros((16,), jnp.float32)
            for c in range(K1 // 16):
                iv = pidx_v[pl.ds(i * K1 + c * 16, 16)]
                acc = acc + plsc.load_gather(tr_v, [iv])
            ls_v[i] = acc
            return carry

        lax.fori_loop(0, LB_W, link_body, 0)
        pltpu.sync_copy(ls_v, loadsum16.at[pl.ds(wid * LB_W, LB_W)])

        def cap_body(c, carry):
            iv = cidx_v[pl.ds(c * 16, 16)]
            capo_v[pl.ds(c * 16, 16)] = plsc.load_gather(cap_v, [iv])
            return carry

        lax.fori_loop(0, CB_W // 16, cap_body, 0)
        pltpu.sync_copy(capo_v, capg.at[pl.ds(wid * CB_W, CB_W)])

    return k


def _relu(x):
    return jnp.maximum(x, 0.0)


def _tc_path_encoder(interpret=False):
    bp = 2000

    def body(f_ref, m_ref, mu_ref, sd_ref, w1_ref, b1_ref,
             w2_ref, b2_ref, out_ref):
        f = (f_ref[...] - mu_ref[...]) / sd_ref[...]
        oh = (m_ref[...] == lax.broadcasted_iota(jnp.int32, (bp, 7), 1))
        oh = oh.astype(jnp.float32)
        x = jnp.concatenate([f[:, 0:2], oh, f[:, 2:10]], axis=1)
        h1 = _relu(x @ w1_ref[...] + b1_ref[...])
        out_ref[...] = _relu(h1 @ w2_ref[...] + b2_ref[...])

    return pl.pallas_call(
        body,
        grid=(P // bp,),
        in_specs=[
            pl.BlockSpec((bp, 10), lambda i: (i, 0)),
            pl.BlockSpec((bp, 1), lambda i: (i, 0)),
            pl.BlockSpec((1, 10), lambda i: (0, 0)),
            pl.BlockSpec((1, 10), lambda i: (0, 0)),
            pl.BlockSpec((17, 32), lambda i: (0, 0)),
            pl.BlockSpec((1, 32), lambda i: (0, 0)),
            pl.BlockSpec((32, 32), lambda i: (0, 0)),
            pl.BlockSpec((1, 32), lambda i: (0, 0)),
        ],
        out_specs=pl.BlockSpec((bp, 32), lambda i: (i, 0)),
        out_shape=jax.ShapeDtypeStruct((P, 32), jnp.float32),
        interpret=interpret,
    )


def _tc_link_encoder(interpret=False):
    def body(ls16_ref, cap_ref, pol_ref, w1_ref, b1_ref, w2_ref,
             b2_ref, wxl_ref, out_ref, outw_ref):
        load = (jnp.sum(ls16_ref[...], axis=1, keepdims=True)
                / cap_ref[...])
        oh = (pol_ref[...] == lax.broadcasted_iota(jnp.int32, (L, 4), 1))
        oh = oh.astype(jnp.float32)
        x = jnp.concatenate([load, oh], axis=1)
        h1 = _relu(x @ w1_ref[...] + b1_ref[...])
        ls = _relu(h1 @ w2_ref[...] + b2_ref[...])
        out_ref[...] = ls
        outw_ref[...] = ls @ wxl_ref[...]

    return pl.pallas_call(
        body,
        in_specs=[pl.BlockSpec((L, 16), lambda: (0, 0)),
                  pl.BlockSpec((L, 1), lambda: (0, 0)),
                  pl.BlockSpec((L, 1), lambda: (0, 0)),
                  pl.BlockSpec((5, 32), lambda: (0, 0)),
                  pl.BlockSpec((1, 32), lambda: (0, 0)),
                  pl.BlockSpec((32, 32), lambda: (0, 0)),
                  pl.BlockSpec((1, 32), lambda: (0, 0)),
                  pl.BlockSpec((32, 32), lambda: (0, 0))],
        out_specs=[pl.BlockSpec((L, 32), lambda: (0, 0)),
                   pl.BlockSpec((L, 32), lambda: (0, 0))],
        out_shape=[jax.ShapeDtypeStruct((L, 32), jnp.float32),
                   jax.ShapeDtypeStruct((L, 32), jnp.float32)],
        interpret=interpret,
    )


def _tc_queue_encoder(interpret=False):
    mu, sdv = ZSC['queue_size']

    def body(qs_ref, pri_ref, w_ref, w1_ref, b1_ref,
             w2_ref, b2_ref, wxq_ref, wxl_ref, out_ref, outw_ref, outlw_ref):
        qs = (qs_ref[...] - mu) / sdv
        oh = (pri_ref[...] == lax.broadcasted_iota(jnp.int32, (Q, 3), 1))
        oh = oh.astype(jnp.float32)
        x = jnp.concatenate([qs, oh, w_ref[...]], axis=1)
        h1 = _relu(x @ w1_ref[...] + b1_ref[...])
        s = _relu(h1 @ w2_ref[...] + b2_ref[...])
        out_ref[...] = s
        outw_ref[...] = s @ wxq_ref[...]
        outlw_ref[...] = s @ wxl_ref[...]

    return pl.pallas_call(
        body,
        in_specs=[pl.BlockSpec((Q, 1), lambda: (0, 0)),
                  pl.BlockSpec((Q, 1), lambda: (0, 0)),
                  pl.BlockSpec((Q, 1), lambda: (0, 0)),
                  pl.BlockSpec((5, 32), lambda: (0, 0)),
                  pl.BlockSpec((1, 32), lambda: (0, 0)),
                  pl.BlockSpec((32, 32), lambda: (0, 0)),
                  pl.BlockSpec((1, 32), lambda: (0, 0)),
                  pl.BlockSpec((32, 32), lambda: (0, 0)),
                  pl.BlockSpec((32, 32), lambda: (0, 0))],
        out_specs=[pl.BlockSpec((Q, 32), lambda: (0, 0)),
                   pl.BlockSpec((Q, 32), lambda: (0, 0)),
                   pl.BlockSpec((Q, 32), lambda: (0, 0))],
        out_shape=[jax.ShapeDtypeStruct((Q, 32), jnp.float32),
                   jax.ShapeDtypeStruct((Q, 32), jnp.float32),
                   jax.ShapeDtypeStruct((Q, 32), jnp.float32)],
        interpret=interpret,
    )


def _tc_path_rnn(interpret=False):
    P4 = P // 4

    def body(qgw_ref, lgw_ref, ps_ref, whb_ref, b_ref, out_ref):
        h = ps_ref[...]
        out_ref[0] = h
        whb, b = whb_ref[...], b_ref[...]
        for t in range(PL):
            h = jnp.tanh(qgw_ref[t] + lgw_ref[t] + h @ whb + b)
            out_ref[t + 1] = h

    return pl.pallas_call(
        body,
        in_specs=[pl.BlockSpec((PL, P4, 128), lambda: (0, 0, 0)),
                  pl.BlockSpec((PL, P4, 128), lambda: (0, 0, 0)),
                  pl.BlockSpec((P4, 128), lambda: (0, 0)),
                  pl.BlockSpec((128, 128), lambda: (0, 0)),
                  pl.BlockSpec((1, 128), lambda: (0, 0))],
        out_specs=pl.BlockSpec((PL + 1, P4, 128), lambda: (0, 0, 0)),
        out_shape=jax.ShapeDtypeStruct((PL + 1, P4, 128), jnp.float32),
        interpret=interpret,
    )


def _tc_queue_update(interpret=False):
    bq = 600

    def body(pg_ref, qs_ref, wx_ref, wh_ref, b_ref, wxq_ref, wxl_ref,
             out_ref, outw_ref, outlw_ref):
        s = jnp.tanh(pg_ref[...] @ wx_ref[...] + qs_ref[...] @ wh_ref[...]
                     + b_ref[...])
        out_ref[...] = s
        outw_ref[...] = s @ wxq_ref[...]
        outlw_ref[...] = s @ wxl_ref[...]

    return pl.pallas_call(
        body,
        grid=(Q // bq,),
        in_specs=[pl.BlockSpec((bq, 32), lambda i: (i, 0)),
                  pl.BlockSpec((bq, 32), lambda i: (i, 0)),
                  pl.BlockSpec((32, 32), lambda i: (0, 0)),
                  pl.BlockSpec((32, 32), lambda i: (0, 0)),
                  pl.BlockSpec((1, 32), lambda i: (0, 0)),
                  pl.BlockSpec((32, 32), lambda i: (0, 0)),
                  pl.BlockSpec((32, 32), lambda i: (0, 0))],
        out_specs=[pl.BlockSpec((bq, 32), lambda i: (i, 0)),
                   pl.BlockSpec((bq, 32), lambda i: (i, 0)),
                   pl.BlockSpec((bq, 32), lambda i: (i, 0))],
        out_shape=[jax.ShapeDtypeStruct((Q, 32), jnp.float32),
                   jax.ShapeDtypeStruct((Q, 32), jnp.float32),
                   jax.ShapeDtypeStruct((Q, 32), jnp.float32)],
        interpret=interpret,
    )


def _tc_link_rnn(interpret=False):
    def body(qgw_ref, ls_ref, wh_ref, b_ref, wxl_ref, out_ref, outw_ref):
        h = ls_ref[...]
        wh, b = wh_ref[...], b_ref[...]
        for t in range(QPL):
            h = jnp.tanh(qgw_ref[:, t, :] + h @ wh + b)
        out_ref[...] = h
        outw_ref[...] = h @ wxl_ref[...]

    return pl.pallas_call(
        body,
        in_specs=[pl.BlockSpec((L, QPL, 32), lambda: (0, 0, 0)),
                  pl.BlockSpec((L, 32), lambda: (0, 0)),
                  pl.BlockSpec((32, 32), lambda: (0, 0)),
                  pl.BlockSpec((1, 32), lambda: (0, 0)),
                  pl.BlockSpec((32, 32), lambda: (0, 0))],
        out_specs=[pl.BlockSpec((L, 32), lambda: (0, 0)),
                   pl.BlockSpec((L, 32), lambda: (0, 0))],
        out_shape=[jax.ShapeDtypeStruct((L, 32), jnp.float32),
                   jax.ShapeDtypeStruct((L, 32), jnp.float32)],
        interpret=interpret,
    )


def _tc_readout(interpret=False):
    bp = 1000

    def body(h_ref, cap_ref, len_ref, tr_ref, pk_ref, w1_ref, b1_ref,
             w2_ref, b2_ref, w3_ref, b3_ref, out_ref):
        w1, b1 = w1_ref[...], b1_ref[...]
        w2, b2 = w2_ref[...], b2_ref[...]
        w3, b3 = w3_ref[...], b3_ref[...]
        ln = len_ref[...]
        qd = jnp.zeros((bp, 1), jnp.float32)
        sic = jnp.zeros((bp, 1), jnp.float32)
        for t in range(PL):
            o = _relu(h_ref[t] @ w1 + b1)
            o = _relu(o @ w2 + b2)
            o = o @ w3 + b3
            ic = 1.0 / cap_ref[:, t:t + 1]
            m = jnp.where(ln > t, 1.0, 0.0)
            qd = qd + m * o * ic
            sic = sic + m * ic
        out_ref[...] = qd + (tr_ref[...] / pk_ref[...]) * sic

    return pl.pallas_call(
        body,
        grid=(P // bp,),
        in_specs=[pl.BlockSpec((PL, bp, 32), lambda i: (0, i, 0)),
                  pl.BlockSpec((bp, PL), lambda i: (i, 0)),
                  pl.BlockSpec((bp, 1), lambda i: (i, 0)),
                  pl.BlockSpec((bp, 1), lambda i: (i, 0)),
                  pl.BlockSpec((bp, 1), lambda i: (i, 0)),
                  pl.BlockSpec((32, 16), lambda i: (0, 0)),
                  pl.BlockSpec((1, 16), lambda i: (0, 0)),
                  pl.BlockSpec((16, 16), lambda i: (0, 0)),
                  pl.BlockSpec((1, 16), lambda i: (0, 0)),
                  pl.BlockSpec((16, 1), lambda i: (0, 0)),
                  pl.BlockSpec((1, 1), lambda i: (0, 0))],
        out_specs=pl.BlockSpec((bp, 1), lambda i: (i, 0)),
        out_shape=jax.ShapeDtypeStruct((P, 1), jnp.float32),
        interpret=interpret,
    )


def _pad_to(x, n):
    return jnp.concatenate(
        [x, jnp.zeros((n - x.shape[0],) + x.shape[1:], x.dtype)], axis=0)


def kernel(traffic, packets, eq_lambda, avg_pkts_lambda, exp_max_factor, pkts_lambda_on, avg_t_off, avg_t_on, ar_a, sigma, capacity, queue_size, weight, W_pe1, b_pe1, W_pe2, b_pe2, W_le1, b_le1, W_le2, b_le2, W_qe1, b_qe1, W_qe2, b_qe2, Wx_p, b_p, Wh_p, Wx_q, b_q, Wh_q, Wx_l, b_l, Wh_l, W_r1, b_r1, W_r2, b_r2, W_r3, b_r3, length, model, policy, priority, queue_to_path, link_to_path, path_to_link, path_to_queue, queue_to_link):
    f32 = jnp.float32
    i32 = jnp.int32

    n_pp = 20
    qtp_idx = _pad_to(queue_to_path.T.reshape(-1).astype(i32),
                      NW * n_pp * CH).reshape(NW, n_pp, CH)
    ltp_idx = _pad_to(link_to_path.T.reshape(-1).astype(i32),
                      NW * n_pp * CH).reshape(NW, n_pp, CH)
    n_q = 24
    p2q_flat = (path_to_queue[:, :, 1] * P
                + path_to_queue[:, :, 0]).reshape(-1).astype(i32)
    p2q_idx = _pad_to(p2q_flat, NW * n_q * CH).reshape(NW, n_q, CH)
    qtl_idx = _pad_to(queue_to_link.reshape(-1).astype(i32),
                      NW * 96).reshape(NW, 1, 96)
    ptl_idx = _pad_to(path_to_link[:, :, 0].reshape(-1).astype(i32),
                      NW * LB_W * K1).reshape(NW, LB_W * K1)
    cap_idx = _pad_to(link_to_path.reshape(-1).astype(i32),
                      CAP_B).reshape(NW, CB_W)

    loadsum16, capg_flat = _sc_prep()(traffic.reshape(-1), capacity.reshape(-1),
                                      ptl_idx, cap_idx)
    capg = capg_flat[:P * PL].reshape(P, PL)

    feats10 = jnp.concatenate([traffic, packets, eq_lambda, avg_pkts_lambda,
                               exp_max_factor, pkts_lambda_on, avg_t_off,
                               avg_t_on, ar_a, sigma], axis=1)
    znames = ('traffic', 'packets', 'eq_lambda', 'avg_pkts_lambda',
              'exp_max_factor', 'pkts_lambda_on', 'avg_t_off', 'avg_t_on',
              'ar_a', 'sigma')
    zmu = jnp.array([[ZSC[n][0] for n in znames]], f32)
    zsd = jnp.array([[ZSC[n][1] for n in znames]], f32)
    Wxq_p, Wxl_p = Wx_p[0:32], Wx_p[32:64]
    path_state = _tc_path_encoder()(
        feats10, model.reshape(P, 1).astype(i32), zmu, zsd,
        W_pe1, b_pe1.reshape(1, 32), W_pe2, b_pe2.reshape(1, 32))
    link_state, link_w = _tc_link_encoder()(
        loadsum16[:L], capacity, policy.reshape(L, 1).astype(i32),
        W_le1, b_le1.reshape(1, 32), W_le2, b_le2.reshape(1, 32), Wxl_p)
    queue_state, queue_w, queue_lw = _tc_queue_encoder()(
        queue_size, priority.reshape(Q, 1).astype(i32), weight,
        W_qe1, b_qe1.reshape(1, 32), W_qe2, b_qe2.reshape(1, 32),
        Wxq_p, Wx_l)

    gather_q = _sc_gather_one(n_pp, CH)
    gather_l = _sc_gather_one(n_pp, CH)
    gather_p2q = _sc_gather_sum(n_q)
    gather_qtl = _sc_gather_one(1, 96)
    path_rnn = _tc_path_rnn()
    queue_update = _tc_queue_update()
    link_rnn = _tc_link_rnn()

    b_p2 = b_p.reshape(1, 32)
    b_q2 = b_q.reshape(1, 32)
    b_l2 = b_l.reshape(1, 32)

    Whp_blk = jax.scipy.linalg.block_diag(Wh_p, Wh_p, Wh_p, Wh_p)
    b_p4 = jnp.tile(b_p, 4).reshape(1, 128)
    ps4 = path_state.reshape(P // 4, 128)

    pss_tm = None
    for _ in range(ITERS):
        qgw_f = gather_q(queue_w, qtp_idx)
        lgw_f = gather_l(link_w, ltp_idx)
        qgw = qgw_f[:P * PL].reshape(PL, P // 4, 128)
        lgw = lgw_f[:P * PL].reshape(PL, P // 4, 128)
        pss_tm = path_rnn(qgw, lgw, ps4, Whp_blk, b_p4)
        ps4 = pss_tm[PL]
        psum = gather_p2q(pss_tm.reshape((PL + 1) * P, 32), p2q_idx)[:Q]
        queue_state, queue_w, queue_lw = queue_update(
            psum, queue_state, Wx_q, Wh_q, b_q2, Wxq_p, Wx_l)
        qglw_f = gather_qtl(queue_lw, qtl_idx)
        qglw = qglw_f[:L * QPL].reshape(L, QPL, 32)
        link_state, link_w = link_rnn(qglw, link_state, Wh_l, b_l2, Wxl_p)

    hidden = pss_tm.reshape(PL + 1, P, 32)[1:]
    out = _tc_readout()(
        hidden, capg, length.reshape(P, 1).astype(i32), traffic, packets,
        W_r1, b_r1.reshape(1, 16), W_r2, b_r2.reshape(1, 16),
        W_r3, b_r3.reshape(1, 1))
    return out


# speedup vs baseline: 1.0551x; 1.0008x over previous
import functools

import jax
import jax.numpy as jnp
from jax import lax
from jax.experimental import pallas as pl
from jax.experimental.pallas import tpu as pltpu
from jax.experimental.pallas import tpu_sc as plsc

P, L, Q = 10000, 1000, 3000
PL = 8
K1, K2, QPL = 80, 32, 3
ITERS = 8
ZSC = {'traffic': [1385.4059, 859.8119], 'packets': [1.4015, 0.8933], 'eq_lambda': [1350.9712, 858.3162], 'avg_pkts_lambda': [0.9117, 0.9724], 'exp_max_factor': [6.6636, 4.7151], 'pkts_lambda_on': [0.9116, 1.6513], 'avg_t_off': [1.6649, 2.3564], 'avg_t_on': [1.6649, 2.3564], 'ar_a': [0.0, 1.0], 'sigma': [0.0, 1.0], 'capacity': [27611.0918, 20090.6211], 'queue_size': [30259.1055, 21410.0957]}

NC, NS = 2, 16
NW = NC * NS
CH = 128


def _sc_mesh():
    return plsc.VectorSubcoreMesh(core_axis_name="c", subcore_axis_name="s",
                                  num_cores=NC, num_subcores=NS)


def _wid():
    return lax.axis_index("s") * NC + lax.axis_index("c")


def _sc_gather_pair(vq, vl, n_chunks, interpret=False):
    b = n_chunks * CH

    @functools.partial(
        pl.kernel,
        out_type=(jax.ShapeDtypeStruct((NW * b, 32), jnp.float32),
                  jax.ShapeDtypeStruct((NW * b, 32), jnp.float32)),
        mesh=_sc_mesh(),
        scratch_types=[pltpu.VMEM((n_chunks, CH), jnp.int32),
                       pltpu.VMEM((b, 32), jnp.float32),
                       pltpu.SemaphoreType.DMA],
        compiler_params=pltpu.CompilerParams(use_tc_tiling_on_sc=False),
        interpret=interpret,
    )
    def k(qtab, ltab, qidx, lidx, qout, lout, idx_v, rows_v, sem):
        wid = _wid()
        base = wid * b
        for tab, idx, out in ((qtab, qidx, qout), (ltab, lidx, lout)):
            pltpu.sync_copy(idx.at[wid], idx_v)
            descs = []
            for c in range(n_chunks):
                descs.append(pltpu.async_copy(
                    tab.at[idx_v.at[c]], rows_v.at[pl.ds(c * CH, CH)], sem))
            for d in descs:
                d.wait()
            pltpu.sync_copy(rows_v, out.at[pl.ds(base, b)])

    return k


def _sc_gather_one(n_chunks, ch, interpret=False):
    b = n_chunks * ch

    @functools.partial(
        pl.kernel,
        out_type=jax.ShapeDtypeStruct((NW * b, 32), jnp.float32),
        mesh=_sc_mesh(),
        scratch_types=[pltpu.VMEM((n_chunks, ch), jnp.int32),
                       pltpu.VMEM((b, 32), jnp.float32),
                       pltpu.SemaphoreType.DMA],
        compiler_params=pltpu.CompilerParams(use_tc_tiling_on_sc=False),
        interpret=interpret,
    )
    def k(tab, idx, out, idx_v, rows_v, sem):
        wid = _wid()
        base = wid * b
        pltpu.sync_copy(idx.at[wid], idx_v)
        descs = []
        for c in range(n_chunks):
            descs.append(pltpu.async_copy(
                tab.at[idx_v.at[c]], rows_v.at[pl.ds(c * ch, ch)], sem))
        for d in descs:
            d.wait()
        pltpu.sync_copy(rows_v, out.at[pl.ds(base, b)])

    return k


def _sc_gather_sum(n_chunks, interpret=False):
    b = n_chunks * CH
    GQ = b // K2

    @functools.partial(
        pl.kernel,
        out_type=jax.ShapeDtypeStruct((NW * GQ, 32), jnp.float32),
        mesh=_sc_mesh(),
        scratch_types=[pltpu.VMEM((n_chunks, CH), jnp.int32),
                       pltpu.VMEM((b, 32), jnp.float32),
                       pltpu.VMEM((GQ, 32), jnp.float32),
                       pltpu.SemaphoreType.DMA],
        compiler_params=pltpu.CompilerParams(use_tc_tiling_on_sc=False,
                                             needs_layout_passes=False),
        interpret=interpret,
    )
    def k(tab, idx, out, idx_v, rows_v, sum_v, sem):
        wid = _wid()
        pltpu.sync_copy(idx.at[wid], idx_v)
        descs = []
        for c in range(n_chunks):
            descs.append(pltpu.async_copy(
                tab.at[idx_v.at[c]], rows_v.at[pl.ds(c * CH, CH)], sem))
        for d in descs:
            d.wait()

        def q_body(q, carry):
            base = q * K2
            a0 = jnp.zeros((16,), jnp.float32)
            a1 = jnp.zeros((16,), jnp.float32)
            for kk in range(K2):
                a0 = a0 + rows_v[base + kk, pl.ds(0, 16)]
                a1 = a1 + rows_v[base + kk, pl.ds(16, 16)]
            sum_v[q, pl.ds(0, 16)] = a0
            sum_v[q, pl.ds(16, 16)] = a1
            return carry

        lax.fori_loop(0, GQ, q_body, 0)
        pltpu.sync_copy(sum_v, out.at[pl.ds(wid * GQ, GQ)])

    return k


LB_W = 32
CB_W = 2512
CAP_B = NW * CB_W


def _sc_prep(interpret=False):
    @functools.partial(
        pl.kernel,
        out_type=(jax.ShapeDtypeStruct((NW * LB_W, 16), jnp.float32),
                  jax.ShapeDtypeStruct((CAP_B,), jnp.float32)),
        mesh=_sc_mesh(),
        scratch_types=[pltpu.VMEM((P,), jnp.float32),
                       pltpu.VMEM((L,), jnp.float32),
                       pltpu.VMEM((LB_W * K1,), jnp.int32),
                       pltpu.VMEM((CB_W,), jnp.int32),
                       pltpu.VMEM((LB_W, 16), jnp.float32),
                       pltpu.VMEM((CB_W,), jnp.float32)],
        compiler_params=pltpu.CompilerParams(needs_layout_passes=False),
        interpret=interpret,
    )
    def k(traffic, cap, ptl_idx, cap_idx, loadsum16, capg,
          tr_v, cap_v, pidx_v, cidx_v, ls_v, capo_v):
        wid = _wid()
        pltpu.sync_copy(traffic, tr_v)
        pltpu.sync_copy(cap, cap_v)
        pltpu.sync_copy(ptl_idx.at[wid], pidx_v)
        pltpu.sync_copy(cap_idx.at[wid], cidx_v)

        def link_body(i, carry):
            acc = jnp.zeros((16,), jnp.float32)
            for c in range(K1 // 16):
                iv = pidx_v[pl.ds(i * K1 + c * 16, 16)]
                acc = acc + plsc.load_gather(tr_v, [iv])
            ls_v[i] = acc
            return carry

        lax.fori_loop(0, LB_W, link_body, 0)
        pltpu.sync_copy(ls_v, loadsum16.at[pl.ds(wid * LB_W, LB_W)])

        def cap_body(c, carry):
            iv = cidx_v[pl.ds(c * 16, 16)]
            capo_v[pl.ds(c * 16, 16)] = plsc.load_gather(cap_v, [iv])
            return carry

        lax.fori_loop(0, CB_W // 16, cap_body, 0)
        pltpu.sync_copy(capo_v, capg.at[pl.ds(wid * CB_W, CB_W)])

    return k


def _relu(x):
    return jnp.maximum(x, 0.0)


def _tc_path_encoder(interpret=False):
    bp = 2000

    def body(f_ref, m_ref, mu_ref, sd_ref, w1_ref, b1_ref,
             w2_ref, b2_ref, out_ref):
        f = (f_ref[...] - mu_ref[...]) / sd_ref[...]
        oh = (m_ref[...] == lax.broadcasted_iota(jnp.int32, (bp, 7), 1))
        oh = oh.astype(jnp.float32)
        x = jnp.concatenate([f[:, 0:2], oh, f[:, 2:10]], axis=1)
        h1 = _relu(x @ w1_ref[...] + b1_ref[...])
        out_ref[...] = _relu(h1 @ w2_ref[...] + b2_ref[...])

    return pl.pallas_call(
        body,
        grid=(P // bp,),
        in_specs=[
            pl.BlockSpec((bp, 10), lambda i: (i, 0)),
            pl.BlockSpec((bp, 1), lambda i: (i, 0)),
            pl.BlockSpec((1, 10), lambda i: (0, 0)),
            pl.BlockSpec((1, 10), lambda i: (0, 0)),
            pl.BlockSpec((17, 32), lambda i: (0, 0)),
            pl.BlockSpec((1, 32), lambda i: (0, 0)),
            pl.BlockSpec((32, 32), lambda i: (0, 0)),
            pl.BlockSpec((1, 32), lambda i: (0, 0)),
        ],
        out_specs=pl.BlockSpec((bp, 32), lambda i: (i, 0)),
        out_shape=jax.ShapeDtypeStruct((P, 32), jnp.float32),
        interpret=interpret,
    )


def _tc_link_encoder(interpret=False):
    def body(ls16_ref, cap_ref, pol_ref, w1_ref, b1_ref, w2_ref,
             b2_ref, wxl_ref, out_ref, outw_ref):
        load = (jnp.sum(ls16_ref[...], axis=1, keepdims=True)
                / cap_ref[...])
        oh = (pol_ref[...] == lax.broadcasted_iota(jnp.int32, (L, 4), 1))
        oh = oh.astype(jnp.float32)
        x = jnp.concatenate([load, oh], axis=1)
        h1 = _relu(x @ w1_ref[...] + b1_ref[...])
        ls = _relu(h1 @ w2_ref[...] + b2_ref[...])
        out_ref[...] = ls
        outw_ref[...] = ls @ wxl_ref[...]

    return pl.pallas_call(
        body,
        in_specs=[pl.BlockSpec((L, 16), lambda: (0, 0)),
                  pl.BlockSpec((L, 1), lambda: (0, 0)),
                  pl.BlockSpec((L, 1), lambda: (0, 0)),
                  pl.BlockSpec((5, 32), lambda: (0, 0)),
                  pl.BlockSpec((1, 32), lambda: (0, 0)),
                  pl.BlockSpec((32, 32), lambda: (0, 0)),
                  pl.BlockSpec((1, 32), lambda: (0, 0)),
                  pl.BlockSpec((32, 32), lambda: (0, 0))],
        out_specs=[pl.BlockSpec((L, 32), lambda: (0, 0)),
                   pl.BlockSpec((L, 32), lambda: (0, 0))],
        out_shape=[jax.ShapeDtypeStruct((L, 32), jnp.float32),
                   jax.ShapeDtypeStruct((L, 32), jnp.float32)],
        interpret=interpret,
    )


def _tc_queue_encoder(interpret=False):
    mu, sdv = ZSC['queue_size']

    def body(qs_ref, pri_ref, w_ref, w1_ref, b1_ref,
             w2_ref, b2_ref, wxq_ref, wxl_ref, out_ref, outw_ref, outlw_ref):
        qs = (qs_ref[...] - mu) / sdv
        oh = (pri_ref[...] == lax.broadcasted_iota(jnp.int32, (Q, 3), 1))
        oh = oh.astype(jnp.float32)
        x = jnp.concatenate([qs, oh, w_ref[...]], axis=1)
        h1 = _relu(x @ w1_ref[...] + b1_ref[...])
        s = _relu(h1 @ w2_ref[...] + b2_ref[...])
        out_ref[...] = s
        outw_ref[...] = s @ wxq_ref[...]
        outlw_ref[...] = s @ wxl_ref[...]

    return pl.pallas_call(
        body,
        in_specs=[pl.BlockSpec((Q, 1), lambda: (0, 0)),
                  pl.BlockSpec((Q, 1), lambda: (0, 0)),
                  pl.BlockSpec((Q, 1), lambda: (0, 0)),
                  pl.BlockSpec((5, 32), lambda: (0, 0)),
                  pl.BlockSpec((1, 32), lambda: (0, 0)),
                  pl.BlockSpec((32, 32), lambda: (0, 0)),
                  pl.BlockSpec((1, 32), lambda: (0, 0)),
                  pl.BlockSpec((32, 32), lambda: (0, 0)),
                  pl.BlockSpec((32, 32), lambda: (0, 0))],
        out_specs=[pl.BlockSpec((Q, 32), lambda: (0, 0)),
                   pl.BlockSpec((Q, 32), lambda: (0, 0)),
                   pl.BlockSpec((Q, 32), lambda: (0, 0))],
        out_shape=[jax.ShapeDtypeStruct((Q, 32), jnp.float32),
                   jax.ShapeDtypeStruct((Q, 32), jnp.float32),
                   jax.ShapeDtypeStruct((Q, 32), jnp.float32)],
        interpret=interpret,
    )


def _tc_path_rnn(interpret=False):
    P4 = P // 4

    def body(qgw_ref, lgw_ref, ps_ref, whb_ref, b_ref, out_ref):
        h = ps_ref[...]
        out_ref[0] = h
        whb, b = whb_ref[...], b_ref[...]
        for t in range(PL):
            h = jnp.tanh(qgw_ref[t] + lgw_ref[t] + h @ whb + b)
            out_ref[t + 1] = h

    return pl.pallas_call(
        body,
        in_specs=[pl.BlockSpec((PL, P4, 128), lambda: (0, 0, 0)),
                  pl.BlockSpec((PL, P4, 128), lambda: (0, 0, 0)),
                  pl.BlockSpec((P4, 128), lambda: (0, 0)),
                  pl.BlockSpec((128, 128), lambda: (0, 0)),
                  pl.BlockSpec((1, 128), lambda: (0, 0))],
        out_specs=pl.BlockSpec((PL + 1, P4, 128), lambda: (0, 0, 0)),
        out_shape=jax.ShapeDtypeStruct((PL + 1, P4, 128), jnp.float32),
        interpret=interpret,
    )


def _tc_queue_update(interpret=False):
    bq = 600

    def body(pg_ref, qs_ref, wx_ref, wh_ref, b_ref, wxq_ref, wxl_ref,
             out_ref, outw_ref, outlw_ref):
        s = jnp.tanh(pg_ref[...] @ wx_ref[...] + qs_ref[...] @ wh_ref[...]
                     + b_ref[...])
        out_ref[...] = s
        outw_ref[...] = s @ wxq_ref[...]
        outlw_ref[...] = s @ wxl_ref[...]

    return pl.pallas_call(
        body,
        grid=(Q // bq,),
        in_specs=[pl.BlockSpec((bq, 32), lambda i: (i, 0)),
                  pl.BlockSpec((bq, 32), lambda i: (i, 0)),
                  pl.BlockSpec((32, 32), lambda i: (0, 0)),
                  pl.BlockSpec((32, 32), lambda i: (0, 0)),
                  pl.BlockSpec((1, 32), lambda i: (0, 0)),
                  pl.BlockSpec((32, 32), lambda i: (0, 0)),
                  pl.BlockSpec((32, 32), lambda i: (0, 0))],
        out_specs=[pl.BlockSpec((bq, 32), lambda i: (i, 0)),
                   pl.BlockSpec((bq, 32), lambda i: (i, 0)),
                   pl.BlockSpec((bq, 32), lambda i: (i, 0))],
        out_shape=[jax.ShapeDtypeStruct((Q, 32), jnp.float32),
                   jax.ShapeDtypeStruct((Q, 32), jnp.float32),
                   jax.ShapeDtypeStruct((Q, 32), jnp.float32)],
        interpret=interpret,
    )


def _tc_link_rnn(interpret=False):
    def body(qgw_ref, ls_ref, wh_ref, b_ref, wxl_ref, out_ref, outw_ref):
        h = ls_ref[...]
        wh, b = wh_ref[...], b_ref[...]
        for t in range(QPL):
            h = jnp.tanh(qgw_ref[:, t, :] + h @ wh + b)
        out_ref[...] = h
        outw_ref[...] = h @ wxl_ref[...]

    return pl.pallas_call(
        body,
        in_specs=[pl.BlockSpec((L, QPL, 32), lambda: (0, 0, 0)),
                  pl.BlockSpec((L, 32), lambda: (0, 0)),
                  pl.BlockSpec((32, 32), lambda: (0, 0)),
                  pl.BlockSpec((1, 32), lambda: (0, 0)),
                  pl.BlockSpec((32, 32), lambda: (0, 0))],
        out_specs=[pl.BlockSpec((L, 32), lambda: (0, 0)),
                   pl.BlockSpec((L, 32), lambda: (0, 0))],
        out_shape=[jax.ShapeDtypeStruct((L, 32), jnp.float32),
                   jax.ShapeDtypeStruct((L, 32), jnp.float32)],
        interpret=interpret,
    )


def _tc_readout(interpret=False):
    bp = 1000

    def body(h_ref, cap_ref, len_ref, tr_ref, pk_ref, w1_ref, b1_ref,
             w2_ref, b2_ref, w3_ref, b3_ref, out_ref):
        w1, b1 = w1_ref[...], b1_ref[...]
        w2, b2 = w2_ref[...], b2_ref[...]
        w3, b3 = w3_ref[...], b3_ref[...]
        ln = len_ref[...]
        qd = jnp.zeros((bp, 1), jnp.float32)
        sic = jnp.zeros((bp, 1), jnp.float32)
        for t in range(PL):
            o = _relu(h_ref[t] @ w1 + b1)
            o = _relu(o @ w2 + b2)
            o = o @ w3 + b3
            ic = 1.0 / cap_ref[:, t:t + 1]
            m = jnp.where(ln > t, 1.0, 0.0)
            qd = qd + m * o * ic
            sic = sic + m * ic
        out_ref[...] = qd + (tr_ref[...] / pk_ref[...]) * sic

    return pl.pallas_call(
        body,
        grid=(P // bp,),
        in_specs=[pl.BlockSpec((PL, bp, 32), lambda i: (0, i, 0)),
                  pl.BlockSpec((bp, PL), lambda i: (i, 0)),
                  pl.BlockSpec((bp, 1), lambda i: (i, 0)),
                  pl.BlockSpec((bp, 1), lambda i: (i, 0)),
                  pl.BlockSpec((bp, 1), lambda i: (i, 0)),
                  pl.BlockSpec((32, 16), lambda i: (0, 0)),
                  pl.BlockSpec((1, 16), lambda i: (0, 0)),
                  pl.BlockSpec((16, 16), lambda i: (0, 0)),
                  pl.BlockSpec((1, 16), lambda i: (0, 0)),
                  pl.BlockSpec((16, 1), lambda i: (0, 0)),
                  pl.BlockSpec((1, 1), lambda i: (0, 0))],
        out_specs=pl.BlockSpec((bp, 1), lambda i: (i, 0)),
        out_shape=jax.ShapeDtypeStruct((P, 1), jnp.float32),
        interpret=interpret,
    )


def _pad_to(x, n):
    return jnp.concatenate(
        [x, jnp.zeros((n - x.shape[0],) + x.shape[1:], x.dtype)], axis=0)


def kernel(traffic, packets, eq_lambda, avg_pkts_lambda, exp_max_factor, pkts_lambda_on, avg_t_off, avg_t_on, ar_a, sigma, capacity, queue_size, weight, W_pe1, b_pe1, W_pe2, b_pe2, W_le1, b_le1, W_le2, b_le2, W_qe1, b_qe1, W_qe2, b_qe2, Wx_p, b_p, Wh_p, Wx_q, b_q, Wh_q, Wx_l, b_l, Wh_l, W_r1, b_r1, W_r2, b_r2, W_r3, b_r3, length, model, policy, priority, queue_to_path, link_to_path, path_to_link, path_to_queue, queue_to_link):
    f32 = jnp.float32
    i32 = jnp.int32

    n_pp = 20
    qtp_idx = _pad_to(queue_to_path.T.reshape(-1).astype(i32),
                      NW * n_pp * CH).reshape(NW, n_pp, CH)
    ltp_idx = _pad_to(link_to_path.T.reshape(-1).astype(i32),
                      NW * n_pp * CH).reshape(NW, n_pp, CH)
    n_q = 24
    p2q_flat = (path_to_queue[:, :, 1] * P
                + path_to_queue[:, :, 0]).reshape(-1).astype(i32)
    p2q_idx = _pad_to(p2q_flat, NW * n_q * CH).reshape(NW, n_q, CH)
    qtl_idx = _pad_to(queue_to_link.reshape(-1).astype(i32),
                      NW * 96).reshape(NW, 1, 96)
    ptl_idx = _pad_to(path_to_link[:, :, 0].reshape(-1).astype(i32),
                      NW * LB_W * K1).reshape(NW, LB_W * K1)
    cap_idx = _pad_to(link_to_path.reshape(-1).astype(i32),
                      CAP_B).reshape(NW, CB_W)

    loadsum16, capg_flat = _sc_prep()(traffic.reshape(-1), capacity.reshape(-1),
                                      ptl_idx, cap_idx)
    capg = capg_flat[:P * PL].reshape(P, PL)

    feats10 = jnp.concatenate([traffic, packets, eq_lambda, avg_pkts_lambda,
                               exp_max_factor, pkts_lambda_on, avg_t_off,
                               avg_t_on, ar_a, sigma], axis=1)
    znames = ('traffic', 'packets', 'eq_lambda', 'avg_pkts_lambda',
              'exp_max_factor', 'pkts_lambda_on', 'avg_t_off', 'avg_t_on',
              'ar_a', 'sigma')
    zmu = jnp.array([[ZSC[n][0] for n in znames]], f32)
    zsd = jnp.array([[ZSC[n][1] for n in znames]], f32)
    Wxq_p, Wxl_p = Wx_p[0:32], Wx_p[32:64]
    path_state = _tc_path_encoder()(
        feats10, model.reshape(P, 1).astype(i32), zmu, zsd,
        W_pe1, b_pe1.reshape(1, 32), W_pe2, b_pe2.reshape(1, 32))
    link_state, link_w = _tc_link_encoder()(
        loadsum16[:L], capacity, policy.reshape(L, 1).astype(i32),
        W_le1, b_le1.reshape(1, 32), W_le2, b_le2.reshape(1, 32), Wxl_p)
    queue_state, queue_w, queue_lw = _tc_queue_encoder()(
        queue_size, priority.reshape(Q, 1).astype(i32), weight,
        W_qe1, b_qe1.reshape(1, 32), W_qe2, b_qe2.reshape(1, 32),
        Wxq_p, Wx_l)

    gather_q = _sc_gather_one(n_pp, CH)
    gather_l = _sc_gather_one(n_pp, CH)
    gather_p2q = _sc_gather_sum(n_q)
    gather_qtl = _sc_gather_one(1, 96)
    path_rnn = _tc_path_rnn()
    queue_update = _tc_queue_update()
    link_rnn = _tc_link_rnn()

    b_p2 = b_p.reshape(1, 32)
    b_q2 = b_q.reshape(1, 32)
    b_l2 = b_l.reshape(1, 32)

    Whp_blk = jax.scipy.linalg.block_diag(Wh_p, Wh_p, Wh_p, Wh_p)
    b_p4 = jnp.tile(b_p, 4).reshape(1, 128)
    ps4 = path_state.reshape(P // 4, 128)

    pss_tm = None
    for it in range(ITERS):
        qgw_f = gather_q(queue_w, qtp_idx)
        lgw_f = gather_l(link_w, ltp_idx)
        qgw = qgw_f[:P * PL].reshape(PL, P // 4, 128)
        lgw = lgw_f[:P * PL].reshape(PL, P // 4, 128)
        pss_tm = path_rnn(qgw, lgw, ps4, Whp_blk, b_p4)
        ps4 = pss_tm[PL]
        if it == ITERS - 1:
            break
        psum = gather_p2q(pss_tm.reshape((PL + 1) * P, 32), p2q_idx)[:Q]
        queue_state, queue_w, queue_lw = queue_update(
            psum, queue_state, Wx_q, Wh_q, b_q2, Wxq_p, Wx_l)
        qglw_f = gather_qtl(queue_lw, qtl_idx)
        qglw = qglw_f[:L * QPL].reshape(L, QPL, 32)
        link_state, link_w = link_rnn(qglw, link_state, Wh_l, b_l2, Wxl_p)

    hidden = pss_tm.reshape(PL + 1, P, 32)[1:]
    out = _tc_readout()(
        hidden, capg, length.reshape(P, 1).astype(i32), traffic, packets,
        W_r1, b_r1.reshape(1, 16), W_r2, b_r2.reshape(1, 16),
        W_r3, b_r3.reshape(1, 1))
    return out


# speedup vs baseline: 1.0587x; 1.0034x over previous
import functools

import jax
import jax.numpy as jnp
from jax import lax
from jax.experimental import pallas as pl
from jax.experimental.pallas import tpu as pltpu
from jax.experimental.pallas import tpu_sc as plsc

P, L, Q = 10000, 1000, 3000
PL = 8
K1, K2, QPL = 80, 32, 3
ITERS = 8
ZSC = {'traffic': [1385.4059, 859.8119], 'packets': [1.4015, 0.8933], 'eq_lambda': [1350.9712, 858.3162], 'avg_pkts_lambda': [0.9117, 0.9724], 'exp_max_factor': [6.6636, 4.7151], 'pkts_lambda_on': [0.9116, 1.6513], 'avg_t_off': [1.6649, 2.3564], 'avg_t_on': [1.6649, 2.3564], 'ar_a': [0.0, 1.0], 'sigma': [0.0, 1.0], 'capacity': [27611.0918, 20090.6211], 'queue_size': [30259.1055, 21410.0957]}

NC, NS = 2, 16
NW = NC * NS
CH = 128


def _sc_mesh():
    return plsc.VectorSubcoreMesh(core_axis_name="c", subcore_axis_name="s",
                                  num_cores=NC, num_subcores=NS)


def _wid():
    return lax.axis_index("s") * NC + lax.axis_index("c")


def _sc_gather_pair(vq, vl, n_chunks, interpret=False):
    b = n_chunks * CH

    @functools.partial(
        pl.kernel,
        out_type=(jax.ShapeDtypeStruct((NW * b, 32), jnp.float32),
                  jax.ShapeDtypeStruct((NW * b, 32), jnp.float32)),
        mesh=_sc_mesh(),
        scratch_types=[pltpu.VMEM((n_chunks, CH), jnp.int32),
                       pltpu.VMEM((b, 32), jnp.float32),
                       pltpu.SemaphoreType.DMA],
        compiler_params=pltpu.CompilerParams(use_tc_tiling_on_sc=False),
        interpret=interpret,
    )
    def k(qtab, ltab, qidx, lidx, qout, lout, idx_v, rows_v, sem):
        wid = _wid()
        base = wid * b
        for tab, idx, out in ((qtab, qidx, qout), (ltab, lidx, lout)):
            pltpu.sync_copy(idx.at[wid], idx_v)
            descs = []
            for c in range(n_chunks):
                descs.append(pltpu.async_copy(
                    tab.at[idx_v.at[c]], rows_v.at[pl.ds(c * CH, CH)], sem))
            for d in descs:
                d.wait()
            pltpu.sync_copy(rows_v, out.at[pl.ds(base, b)])

    return k


def _sc_gather_one(n_chunks, ch, interpret=False):
    b = n_chunks * ch

    @functools.partial(
        pl.kernel,
        out_type=jax.ShapeDtypeStruct((NW * b, 32), jnp.float32),
        mesh=_sc_mesh(),
        scratch_types=[pltpu.VMEM((n_chunks, ch), jnp.int32),
                       pltpu.VMEM((b, 32), jnp.float32),
                       pltpu.SemaphoreType.DMA],
        compiler_params=pltpu.CompilerParams(use_tc_tiling_on_sc=False),
        interpret=interpret,
    )
    def k(tab, idx, out, idx_v, rows_v, sem):
        wid = _wid()
        base = wid * b
        pltpu.sync_copy(idx.at[wid], idx_v)
        descs = []
        for c in range(n_chunks):
            descs.append(pltpu.async_copy(
                tab.at[idx_v.at[c]], rows_v.at[pl.ds(c * ch, ch)], sem))
        for d in descs:
            d.wait()
        pltpu.sync_copy(rows_v, out.at[pl.ds(base, b)])

    return k


def _sc_gather_q_qtl(n_chunks, interpret=False):
    b = n_chunks * CH

    @functools.partial(
        pl.kernel,
        out_type=(jax.ShapeDtypeStruct((NW * b, 32), jnp.float32),
                  jax.ShapeDtypeStruct((NW * 96, 32), jnp.float32)),
        mesh=_sc_mesh(),
        scratch_types=[pltpu.VMEM((n_chunks, CH), jnp.int32),
                       pltpu.VMEM((b, 32), jnp.float32),
                       pltpu.VMEM((1, 96), jnp.int32),
                       pltpu.VMEM((96, 32), jnp.float32),
                       pltpu.SemaphoreType.DMA],
        compiler_params=pltpu.CompilerParams(use_tc_tiling_on_sc=False),
        interpret=interpret,
    )
    def k(qtab, qltab, qidx, tidx, qout, tout, idx_v, rows_v, idx2_v,
          rows2_v, sem):
        wid = _wid()
        base = wid * b
        pltpu.sync_copy(tidx.at[wid], idx2_v)
        d2 = pltpu.async_copy(qltab.at[idx2_v.at[0]], rows2_v, sem)
        pltpu.sync_copy(qidx.at[wid], idx_v)
        descs = []
        for c in range(n_chunks):
            descs.append(pltpu.async_copy(
                qtab.at[idx_v.at[c]], rows_v.at[pl.ds(c * CH, CH)], sem))
        d2.wait()
        pltpu.sync_copy(rows2_v, tout.at[pl.ds(wid * 96, 96)])
        for d in descs:
            d.wait()
        pltpu.sync_copy(rows_v, qout.at[pl.ds(base, b)])

    return k


def _sc_gather_sum(n_chunks, interpret=False):
    b = n_chunks * CH
    GQ = b // K2

    @functools.partial(
        pl.kernel,
        out_type=jax.ShapeDtypeStruct((NW * GQ, 32), jnp.float32),
        mesh=_sc_mesh(),
        scratch_types=[pltpu.VMEM((n_chunks, CH), jnp.int32),
                       pltpu.VMEM((b, 32), jnp.float32),
                       pltpu.VMEM((GQ, 32), jnp.float32),
                       pltpu.SemaphoreType.DMA],
        compiler_params=pltpu.CompilerParams(use_tc_tiling_on_sc=False,
                                             needs_layout_passes=False),
        interpret=interpret,
    )
    def k(tab, idx, out, idx_v, rows_v, sum_v, sem):
        wid = _wid()
        pltpu.sync_copy(idx.at[wid], idx_v)
        descs = []
        for c in range(n_chunks):
            descs.append(pltpu.async_copy(
                tab.at[idx_v.at[c]], rows_v.at[pl.ds(c * CH, CH)], sem))
        for d in descs:
            d.wait()

        def q_body(q, carry):
            base = q * K2
            a0 = jnp.zeros((16,), jnp.float32)
            a1 = jnp.zeros((16,), jnp.float32)
            for kk in range(K2):
                a0 = a0 + rows_v[base + kk, pl.ds(0, 16)]
                a1 = a1 + rows_v[base + kk, pl.ds(16, 16)]
            sum_v[q, pl.ds(0, 16)] = a0
            sum_v[q, pl.ds(16, 16)] = a1
            return carry

        lax.fori_loop(0, GQ, q_body, 0)
        pltpu.sync_copy(sum_v, out.at[pl.ds(wid * GQ, GQ)])

    return k


LB_W = 32
CB_W = 2512
CAP_B = NW * CB_W


def _sc_prep(interpret=False):
    @functools.partial(
        pl.kernel,
        out_type=(jax.ShapeDtypeStruct((NW * LB_W, 16), jnp.float32),
                  jax.ShapeDtypeStruct((CAP_B,), jnp.float32)),
        mesh=_sc_mesh(),
        scratch_types=[pltpu.VMEM((P,), jnp.float32),
                       pltpu.VMEM((L,), jnp.float32),
                       pltpu.VMEM((LB_W * K1,), jnp.int32),
                       pltpu.VMEM((CB_W,), jnp.int32),
                       pltpu.VMEM((LB_W, 16), jnp.float32),
                       pltpu.VMEM((CB_W,), jnp.float32)],
        compiler_params=pltpu.CompilerParams(needs_layout_passes=False),
        interpret=interpret,
    )
    def k(traffic, cap, ptl_idx, cap_idx, loadsum16, capg,
          tr_v, cap_v, pidx_v, cidx_v, ls_v, capo_v):
        wid = _wid()
        pltpu.sync_copy(traffic, tr_v)
        pltpu.sync_copy(cap, cap_v)
        pltpu.sync_copy(ptl_idx.at[wid], pidx_v)
        pltpu.sync_copy(cap_idx.at[wid], cidx_v)

        def link_body(i, carry):
            acc = jnp.zeros((16,), jnp.float32)
            for c in range(K1 // 16):
                iv = pidx_v[pl.ds(i * K1 + c * 16, 16)]
                acc = acc + plsc.load_gather(tr_v, [iv])
            ls_v[i] = acc
            return carry

        lax.fori_loop(0, LB_W, link_body, 0)
        pltpu.sync_copy(ls_v, loadsum16.at[pl.ds(wid * LB_W, LB_W)])

        def cap_body(c, carry):
            iv = cidx_v[pl.ds(c * 16, 16)]
            capo_v[pl.ds(c * 16, 16)] = plsc.load_gather(cap_v, [iv])
            return carry

        lax.fori_loop(0, CB_W // 16, cap_body, 0)
        pltpu.sync_copy(capo_v, capg.at[pl.ds(wid * CB_W, CB_W)])

    return k


def _relu(x):
    return jnp.maximum(x, 0.0)


def _tc_path_encoder(interpret=False):
    bp = 2000

    def body(f_ref, m_ref, mu_ref, sd_ref, w1_ref, b1_ref,
             w2_ref, b2_ref, out_ref):
        f = (f_ref[...] - mu_ref[...]) / sd_ref[...]
        oh = (m_ref[...] == lax.broadcasted_iota(jnp.int32, (bp, 7), 1))
        oh = oh.astype(jnp.float32)
        x = jnp.concatenate([f[:, 0:2], oh, f[:, 2:10]], axis=1)
        h1 = _relu(x @ w1_ref[...] + b1_ref[...])
        out_ref[...] = _relu(h1 @ w2_ref[...] + b2_ref[...])

    return pl.pallas_call(
        body,
        grid=(P // bp,),
        in_specs=[
            pl.BlockSpec((bp, 10), lambda i: (i, 0)),
            pl.BlockSpec((bp, 1), lambda i: (i, 0)),
            pl.BlockSpec((1, 10), lambda i: (0, 0)),
            pl.BlockSpec((1, 10), lambda i: (0, 0)),
            pl.BlockSpec((17, 32), lambda i: (0, 0)),
            pl.BlockSpec((1, 32), lambda i: (0, 0)),
            pl.BlockSpec((32, 32), lambda i: (0, 0)),
            pl.BlockSpec((1, 32), lambda i: (0, 0)),
        ],
        out_specs=pl.BlockSpec((bp, 32), lambda i: (i, 0)),
        out_shape=jax.ShapeDtypeStruct((P, 32), jnp.float32),
        interpret=interpret,
    )


def _tc_link_encoder(interpret=False):
    def body(ls16_ref, cap_ref, pol_ref, w1_ref, b1_ref, w2_ref,
             b2_ref, wxl_ref, out_ref, outw_ref):
        load = (jnp.sum(ls16_ref[...], axis=1, keepdims=True)
                / cap_ref[...])
        oh = (pol_ref[...] == lax.broadcasted_iota(jnp.int32, (L, 4), 1))
        oh = oh.astype(jnp.float32)
        x = jnp.concatenate([load, oh], axis=1)
        h1 = _relu(x @ w1_ref[...] + b1_ref[...])
        ls = _relu(h1 @ w2_ref[...] + b2_ref[...])
        out_ref[...] = ls
        outw_ref[...] = ls @ wxl_ref[...]

    return pl.pallas_call(
        body,
        in_specs=[pl.BlockSpec((L, 16), lambda: (0, 0)),
                  pl.BlockSpec((L, 1), lambda: (0, 0)),
                  pl.BlockSpec((L, 1), lambda: (0, 0)),
                  pl.BlockSpec((5, 32), lambda: (0, 0)),
                  pl.BlockSpec((1, 32), lambda: (0, 0)),
                  pl.BlockSpec((32, 32), lambda: (0, 0)),
                  pl.BlockSpec((1, 32), lambda: (0, 0)),
                  pl.BlockSpec((32, 32), lambda: (0, 0))],
        out_specs=[pl.BlockSpec((L, 32), lambda: (0, 0)),
                   pl.BlockSpec((L, 32), lambda: (0, 0))],
        out_shape=[jax.ShapeDtypeStruct((L, 32), jnp.float32),
                   jax.ShapeDtypeStruct((L, 32), jnp.float32)],
        interpret=interpret,
    )


def _tc_queue_encoder(interpret=False):
    mu, sdv = ZSC['queue_size']

    def body(qs_ref, pri_ref, w_ref, w1_ref, b1_ref,
             w2_ref, b2_ref, wxq_ref, wxl_ref, out_ref, outw_ref, outlw_ref):
        qs = (qs_ref[...] - mu) / sdv
        oh = (pri_ref[...] == lax.broadcasted_iota(jnp.int32, (Q, 3), 1))
        oh = oh.astype(jnp.float32)
        x = jnp.concatenate([qs, oh, w_ref[...]], axis=1)
        h1 = _relu(x @ w1_ref[...] + b1_ref[...])
        s = _relu(h1 @ w2_ref[...] + b2_ref[...])
        out_ref[...] = s
        outw_ref[...] = s @ wxq_ref[...]
        outlw_ref[...] = s @ wxl_ref[...]

    return pl.pallas_call(
        body,
        in_specs=[pl.BlockSpec((Q, 1), lambda: (0, 0)),
                  pl.BlockSpec((Q, 1), lambda: (0, 0)),
                  pl.BlockSpec((Q, 1), lambda: (0, 0)),
                  pl.BlockSpec((5, 32), lambda: (0, 0)),
                  pl.BlockSpec((1, 32), lambda: (0, 0)),
                  pl.BlockSpec((32, 32), lambda: (0, 0)),
                  pl.BlockSpec((1, 32), lambda: (0, 0)),
                  pl.BlockSpec((32, 32), lambda: (0, 0)),
                  pl.BlockSpec((32, 32), lambda: (0, 0))],
        out_specs=[pl.BlockSpec((Q, 32), lambda: (0, 0)),
                   pl.BlockSpec((Q, 32), lambda: (0, 0)),
                   pl.BlockSpec((Q, 32), lambda: (0, 0))],
        out_shape=[jax.ShapeDtypeStruct((Q, 32), jnp.float32),
                   jax.ShapeDtypeStruct((Q, 32), jnp.float32),
                   jax.ShapeDtypeStruct((Q, 32), jnp.float32)],
        interpret=interpret,
    )


def _tc_path_rnn(interpret=False):
    P4 = P // 4

    def body(qgw_ref, lgw_ref, ps_ref, whb_ref, b_ref, out_ref):
        h = ps_ref[...]
        out_ref[0] = h
        whb, b = whb_ref[...], b_ref[...]
        for t in range(PL):
            h = jnp.tanh(qgw_ref[t] + lgw_ref[t] + h @ whb + b)
            out_ref[t + 1] = h

    return pl.pallas_call(
        body,
        in_specs=[pl.BlockSpec((PL, P4, 128), lambda: (0, 0, 0)),
                  pl.BlockSpec((PL, P4, 128), lambda: (0, 0, 0)),
                  pl.BlockSpec((P4, 128), lambda: (0, 0)),
                  pl.BlockSpec((128, 128), lambda: (0, 0)),
                  pl.BlockSpec((1, 128), lambda: (0, 0))],
        out_specs=pl.BlockSpec((PL + 1, P4, 128), lambda: (0, 0, 0)),
        out_shape=jax.ShapeDtypeStruct((PL + 1, P4, 128), jnp.float32),
        interpret=interpret,
    )


def _tc_queue_update(interpret=False):
    bq = 600

    def body(pg_ref, qs_ref, wx_ref, wh_ref, b_ref, wxq_ref, wxl_ref,
             out_ref, outw_ref, outlw_ref):
        s = jnp.tanh(pg_ref[...] @ wx_ref[...] + qs_ref[...] @ wh_ref[...]
                     + b_ref[...])
        out_ref[...] = s
        outw_ref[...] = s @ wxq_ref[...]
        outlw_ref[...] = s @ wxl_ref[...]

    return pl.pallas_call(
        body,
        grid=(Q // bq,),
        in_specs=[pl.BlockSpec((bq, 32), lambda i: (i, 0)),
                  pl.BlockSpec((bq, 32), lambda i: (i, 0)),
                  pl.BlockSpec((32, 32), lambda i: (0, 0)),
                  pl.BlockSpec((32, 32), lambda i: (0, 0)),
                  pl.BlockSpec((1, 32), lambda i: (0, 0)),
                  pl.BlockSpec((32, 32), lambda i: (0, 0)),
                  pl.BlockSpec((32, 32), lambda i: (0, 0))],
        out_specs=[pl.BlockSpec((bq, 32), lambda i: (i, 0)),
                   pl.BlockSpec((bq, 32), lambda i: (i, 0)),
                   pl.BlockSpec((bq, 32), lambda i: (i, 0))],
        out_shape=[jax.ShapeDtypeStruct((Q, 32), jnp.float32),
                   jax.ShapeDtypeStruct((Q, 32), jnp.float32),
                   jax.ShapeDtypeStruct((Q, 32), jnp.float32)],
        interpret=interpret,
    )


def _tc_link_rnn(interpret=False):
    def body(qgw_ref, ls_ref, wh_ref, b_ref, wxl_ref, out_ref, outw_ref):
        h = ls_ref[...]
        wh, b = wh_ref[...], b_ref[...]
        for t in range(QPL):
            h = jnp.tanh(qgw_ref[:, t, :] + h @ wh + b)
        out_ref[...] = h
        outw_ref[...] = h @ wxl_ref[...]

    return pl.pallas_call(
        body,
        in_specs=[pl.BlockSpec((L, QPL, 32), lambda: (0, 0, 0)),
                  pl.BlockSpec((L, 32), lambda: (0, 0)),
                  pl.BlockSpec((32, 32), lambda: (0, 0)),
                  pl.BlockSpec((1, 32), lambda: (0, 0)),
                  pl.BlockSpec((32, 32), lambda: (0, 0))],
        out_specs=[pl.BlockSpec((L, 32), lambda: (0, 0)),
                   pl.BlockSpec((L, 32), lambda: (0, 0))],
        out_shape=[jax.ShapeDtypeStruct((L, 32), jnp.float32),
                   jax.ShapeDtypeStruct((L, 32), jnp.float32)],
        interpret=interpret,
    )


def _tc_readout(interpret=False):
    bp = 1000

    def body(h_ref, cap_ref, len_ref, tr_ref, pk_ref, w1_ref, b1_ref,
             w2_ref, b2_ref, w3_ref, b3_ref, out_ref):
        w1, b1 = w1_ref[...], b1_ref[...]
        w2, b2 = w2_ref[...], b2_ref[...]
        w3, b3 = w3_ref[...], b3_ref[...]
        ln = len_ref[...]
        qd = jnp.zeros((bp, 1), jnp.float32)
        sic = jnp.zeros((bp, 1), jnp.float32)
        for t in range(PL):
            o = _relu(h_ref[t] @ w1 + b1)
            o = _relu(o @ w2 + b2)
            o = o @ w3 + b3
            ic = 1.0 / cap_ref[:, t:t + 1]
            m = jnp.where(ln > t, 1.0, 0.0)
            qd = qd + m * o * ic
            sic = sic + m * ic
        out_ref[...] = qd + (tr_ref[...] / pk_ref[...]) * sic

    return pl.pallas_call(
        body,
        grid=(P // bp,),
        in_specs=[pl.BlockSpec((PL, bp, 32), lambda i: (0, i, 0)),
                  pl.BlockSpec((bp, PL), lambda i: (i, 0)),
                  pl.BlockSpec((bp, 1), lambda i: (i, 0)),
                  pl.BlockSpec((bp, 1), lambda i: (i, 0)),
                  pl.BlockSpec((bp, 1), lambda i: (i, 0)),
                  pl.BlockSpec((32, 16), lambda i: (0, 0)),
                  pl.BlockSpec((1, 16), lambda i: (0, 0)),
                  pl.BlockSpec((16, 16), lambda i: (0, 0)),
                  pl.BlockSpec((1, 16), lambda i: (0, 0)),
                  pl.BlockSpec((16, 1), lambda i: (0, 0)),
                  pl.BlockSpec((1, 1), lambda i: (0, 0))],
        out_specs=pl.BlockSpec((bp, 1), lambda i: (i, 0)),
        out_shape=jax.ShapeDtypeStruct((P, 1), jnp.float32),
        interpret=interpret,
    )


def _pad_to(x, n):
    return jnp.concatenate(
        [x, jnp.zeros((n - x.shape[0],) + x.shape[1:], x.dtype)], axis=0)


def kernel(traffic, packets, eq_lambda, avg_pkts_lambda, exp_max_factor, pkts_lambda_on, avg_t_off, avg_t_on, ar_a, sigma, capacity, queue_size, weight, W_pe1, b_pe1, W_pe2, b_pe2, W_le1, b_le1, W_le2, b_le2, W_qe1, b_qe1, W_qe2, b_qe2, Wx_p, b_p, Wh_p, Wx_q, b_q, Wh_q, Wx_l, b_l, Wh_l, W_r1, b_r1, W_r2, b_r2, W_r3, b_r3, length, model, policy, priority, queue_to_path, link_to_path, path_to_link, path_to_queue, queue_to_link):
    f32 = jnp.float32
    i32 = jnp.int32

    n_pp = 20
    qtp_idx = _pad_to(queue_to_path.T.reshape(-1).astype(i32),
                      NW * n_pp * CH).reshape(NW, n_pp, CH)
    ltp_idx = _pad_to(link_to_path.T.reshape(-1).astype(i32),
                      NW * n_pp * CH).reshape(NW, n_pp, CH)
    n_q = 24
    p2q_flat = (path_to_queue[:, :, 1] * P
                + path_to_queue[:, :, 0]).reshape(-1).astype(i32)
    p2q_idx = _pad_to(p2q_flat, NW * n_q * CH).reshape(NW, n_q, CH)
    qtl_idx = _pad_to(queue_to_link.reshape(-1).astype(i32),
                      NW * 96).reshape(NW, 1, 96)
    ptl_idx = _pad_to(path_to_link[:, :, 0].reshape(-1).astype(i32),
                      NW * LB_W * K1).reshape(NW, LB_W * K1)
    cap_idx = _pad_to(link_to_path.reshape(-1).astype(i32),
                      CAP_B).reshape(NW, CB_W)

    loadsum16, capg_flat = _sc_prep()(traffic.reshape(-1), capacity.reshape(-1),
                                      ptl_idx, cap_idx)
    capg = capg_flat[:P * PL].reshape(P, PL)

    feats10 = jnp.concatenate([traffic, packets, eq_lambda, avg_pkts_lambda,
                               exp_max_factor, pkts_lambda_on, avg_t_off,
                               avg_t_on, ar_a, sigma], axis=1)
    znames = ('traffic', 'packets', 'eq_lambda', 'avg_pkts_lambda',
              'exp_max_factor', 'pkts_lambda_on', 'avg_t_off', 'avg_t_on',
              'ar_a', 'sigma')
    zmu = jnp.array([[ZSC[n][0] for n in znames]], f32)
    zsd = jnp.array([[ZSC[n][1] for n in znames]], f32)
    Wxq_p, Wxl_p = Wx_p[0:32], Wx_p[32:64]
    path_state = _tc_path_encoder()(
        feats10, model.reshape(P, 1).astype(i32), zmu, zsd,
        W_pe1, b_pe1.reshape(1, 32), W_pe2, b_pe2.reshape(1, 32))
    link_state, link_w = _tc_link_encoder()(
        loadsum16[:L], capacity, policy.reshape(L, 1).astype(i32),
        W_le1, b_le1.reshape(1, 32), W_le2, b_le2.reshape(1, 32), Wxl_p)
    queue_state, queue_w, queue_lw = _tc_queue_encoder()(
        queue_size, priority.reshape(Q, 1).astype(i32), weight,
        W_qe1, b_qe1.reshape(1, 32), W_qe2, b_qe2.reshape(1, 32),
        Wxq_p, Wx_l)

    gather_q = _sc_gather_one(n_pp, CH)
    gather_q_qtl = _sc_gather_q_qtl(n_pp)
    gather_l = _sc_gather_one(n_pp, CH)
    gather_p2q = _sc_gather_sum(n_q)
    path_rnn = _tc_path_rnn()
    queue_update = _tc_queue_update()
    link_rnn = _tc_link_rnn()

    b_p2 = b_p.reshape(1, 32)
    b_q2 = b_q.reshape(1, 32)
    b_l2 = b_l.reshape(1, 32)

    Whp_blk = jax.scipy.linalg.block_diag(Wh_p, Wh_p, Wh_p, Wh_p)
    b_p4 = jnp.tile(b_p, 4).reshape(1, 128)
    ps4 = path_state.reshape(P // 4, 128)

    pss_tm = None
    qgw_f = gather_q(queue_w, qtp_idx)
    for it in range(ITERS):
        lgw_f = gather_l(link_w, ltp_idx)
        qgw = qgw_f[:P * PL].reshape(PL, P // 4, 128)
        lgw = lgw_f[:P * PL].reshape(PL, P // 4, 128)
        pss_tm = path_rnn(qgw, lgw, ps4, Whp_blk, b_p4)
        ps4 = pss_tm[PL]
        if it == ITERS - 1:
            break
        psum = gather_p2q(pss_tm.reshape((PL + 1) * P, 32), p2q_idx)[:Q]
        queue_state, queue_w, queue_lw = queue_update(
            psum, queue_state, Wx_q, Wh_q, b_q2, Wxq_p, Wx_l)
        qgw_f, qglw_f = gather_q_qtl(queue_w, queue_lw, qtp_idx, qtl_idx)
        qglw = qglw_f[:L * QPL].reshape(L, QPL, 32)
        link_state, link_w = link_rnn(qglw, link_state, Wh_l, b_l2, Wxl_p)

    hidden = pss_tm.reshape(PL + 1, P, 32)[1:]
    out = _tc_readout()(
        hidden, capg, length.reshape(P, 1).astype(i32), traffic, packets,
        W_r1, b_r1.reshape(1, 16), W_r2, b_r2.reshape(1, 16),
        W_r3, b_r3.reshape(1, 1))
    return out
